# Initial kernel scaffold; baseline (speedup 1.0000x reference)
import jax
import jax.numpy as jnp
from jax.experimental import pallas as pl


def kernel(x, Wqk, Wv, Wout, bout, rotations):
    raise NotImplementedError("write your pallas kernel here")


# speedup vs baseline: 1.6601x; 1.6601x over previous
import functools

import jax
import jax.numpy as jnp
from jax import lax
from jax.experimental import pallas as pl
from jax.experimental.pallas import tpu as pltpu

_HEADS = 8
_BUCKET = 64
_NHASH = 8
_NB = 10
_N = 640
_DH = 128
_DIM = 1024
_NSEQ = _NHASH * _N
_NCH = _NSEQ // _BUCKET

_HIGH = lax.Precision.HIGHEST


def _dot16(a, b, dims):
    return lax.dot_general(a.astype(jnp.bfloat16), b.astype(jnp.bfloat16),
                           dims, preferred_element_type=jnp.float32)


def _frontend_body(x_ref, o_ref):
    f32 = jnp.float32
    x = x_ref[0]
    C, H, W = x.shape
    h2, w2 = H // 2, W // 2
    wi = lax.broadcasted_iota(jnp.int32, (W, w2), 0)
    ji = lax.broadcasted_iota(jnp.int32, (W, w2), 1)
    P0 = (wi == 2 * ji).astype(f32)
    P1 = (wi == 2 * ji + 1).astype(f32)
    dn2 = (((1,), (0,)), ((), ()))

    ninf = jnp.full((C, 1, W), -jnp.inf, f32)
    up = jnp.concatenate([x[:, 1:, :], ninf], axis=1)
    dnm = jnp.concatenate([ninf, x[:, :-1, :]], axis=1)
    vm = jnp.maximum(x, jnp.maximum(up, dnm))
    vme = vm.reshape(C * h2, 2, W)[:, 0, :]
    ninfc = jnp.full((C * h2, 1), -jnp.inf, f32)
    lf = jnp.concatenate([vme[:, 1:], ninfc], axis=1)
    rt = jnp.concatenate([ninfc, vme[:, :-1]], axis=1)
    hm = jnp.maximum(vme, jnp.maximum(lf, rt))
    xs = lax.dot_general(hm, P0, dn2, precision=lax.Precision.HIGHEST)

    xr = x.reshape(C * h2, 2, W)
    xe = xr[:, 0, :]
    xo = xr[:, 1, :]
    x00 = lax.dot_general(xe, P0, dn2, precision=lax.Precision.HIGHEST)
    x01 = lax.dot_general(xe, P1, dn2, precision=lax.Precision.HIGHEST)
    x10 = lax.dot_general(xo, P0, dn2, precision=lax.Precision.HIGHEST)
    x11 = lax.dot_general(xo, P1, dn2, precision=lax.Precision.HIGHEST)
    ll = (x00 + x01 + x10 + x11) * 0.5
    lh = (x10 + x11 - x00 - x01) * 0.5
    hl = (x01 + x11 - x00 - x10) * 0.5
    hh = (x00 - x01 - x10 + x11) * 0.5
    o_ref[0] = jnp.concatenate(
        [t.reshape(C, h2, w2) for t in (xs, ll, lh, hl, hh)], axis=0)


def _attn_body(xf_ref, wqk_ref, wv_ref, rot_ref, o_ref,
               sx_ref, skn_ref, st_ref, so_ref, lse_ref, pos_ref, uo_ref,
               ulse_ref):
    f32 = jnp.float32
    xfb = xf_ref[0]
    qk = _dot16(xfb, wqk_ref[...], (((1,), (0,)), ((), ())))
    v = _dot16(xfb, wv_ref[...], (((1,), (0,)), ((), ())))
    X = jnp.concatenate([qk, v], axis=1)
    rot = _dot16(qk, rot_ref[...], (((1,), (0,)), ((), ())))

    ri = lax.broadcasted_iota(jnp.int32, (_N, _N), 0)
    ci = lax.broadcasted_iota(jnp.int32, (_N, _N), 1)
    Tlow = (ci <= ri).astype(f32)
    cif = ci.astype(f32)
    ti = lax.broadcasted_iota(jnp.int32, (_N, 1), 0).astype(f32)

    ki = lax.broadcasted_iota(jnp.int32, (_N, _NB), 1)
    j10 = lax.broadcasted_iota(jnp.int32, (_NB, _NB), 0)
    k10 = lax.broadcasted_iota(jnp.int32, (_NB, _NB), 1)
    Lstrict = (j10 < k10).astype(f32)

    for h in range(_NHASH):
        r5 = rot[:, h * 5:(h + 1) * 5]
        rh = jnp.concatenate([r5, -r5], axis=1)
        mx = jnp.max(rh, axis=1, keepdims=True)
        b = jnp.min(jnp.where(rh >= mx, ki, _NB), axis=1, keepdims=True)
        O = (ki == b).astype(f32)
        colcum = lax.dot_general(Tlow, O, (((1,), (0,)), ((), ())), precision=_HIGH)
        cnt = colcum[_N - 1:_N, :]
        offs = lax.dot_general(cnt, Lstrict, (((1,), (0,)), ((), ())), precision=_HIGH)
        rank = jnp.sum(O * colcum, axis=1, keepdims=True) - 1.0
        base = lax.dot_general(O, offs, (((1,), (1,)), ((), ())), precision=_HIGH)
        pos = base + rank
        pos_ref[:, h:h + 1] = pos
        S = (pos == cif).astype(f32)
        sx = lax.dot_general(S, X, (((0,), (0,)), ((), ())), precision=_HIGH)
        sid = lax.dot_general(S, ti, (((0,), (0,)), ((), ())), precision=_HIGH)
        sk = sx[:, :_DH]
        nrm = jnp.sqrt(jnp.sum(sk * sk, axis=1, keepdims=True))
        kn = sk / jnp.maximum(nrm, 1e-12)
        sx_ref[h * _N:(h + 1) * _N, :] = sx
        skn_ref[h * _N:(h + 1) * _N, :] = kn
        st_ref[h * _N:(h + 1) * _N, :] = sid

    scale = _DH ** -0.5
    ones_q = jnp.ones((_BUCKET, 1), f32)
    ones_k = jnp.ones((2 * _BUCKET, 1), f32)

    def chunk(c, carry):
        pc = lax.rem(c + _NCH - 1, _NCH)
        cur = sx_ref[pl.ds(c * _BUCKET, _BUCKET), :]
        q = cur[:, :_DH]
        kc = skn_ref[pl.ds(c * _BUCKET, _BUCKET), :]
        kp = skn_ref[pl.ds(pc * _BUCKET, _BUCKET), :]
        kk = jnp.concatenate([kc, kp], axis=0)
        prev = sx_ref[pl.ds(pc * _BUCKET, _BUCKET), :]
        vv = jnp.concatenate([cur[:, _DH:], prev[:, _DH:]], axis=0)
        idq = st_ref[pl.ds(c * _BUCKET, _BUCKET), :]
        idk = jnp.concatenate([st_ref[pl.ds(c * _BUCKET, _BUCKET), :],
                               st_ref[pl.ds(pc * _BUCKET, _BUCKET), :]],
                              axis=0)
        dots = _dot16(q, kk, (((1,), (1,)), ((), ()))) * scale
        diff = lax.dot_general(jnp.concatenate([idq, ones_q], axis=1),
                               jnp.concatenate([ones_k, -idk], axis=1),
                               (((1,), (1,)), ((), ())), precision=_HIGH)
        dots = jnp.where(diff == 0.0, -5e4, dots)
        m = jnp.max(dots, axis=1, keepdims=True)
        ex = jnp.exp(dots - m)
        s = jnp.sum(ex, axis=1, keepdims=True)
        so = _dot16(ex / s, vv, (((1,), (0,)), ((), ())))
        so_ref[pl.ds(c * _BUCKET, _BUCKET), :] = so
        lse_ref[pl.ds(c * _BUCKET, _BUCKET), :] = m + jnp.log(s)
        return carry

    lax.fori_loop(0, _NCH, chunk, 0)

    for h in range(_NHASH):
        pos = pos_ref[:, h:h + 1]
        U = (pos == cif).astype(f32)
        seg = jnp.concatenate([so_ref[h * _N:(h + 1) * _N, :],
                               lse_ref[h * _N:(h + 1) * _N, :]], axis=1)
        uo = lax.dot_general(U, seg, (((1,), (0,)), ((), ())), precision=_HIGH)
        uo_ref[:, h * _DH:(h + 1) * _DH] = uo[:, :_DH]
        ulse_ref[:, h:h + 1] = uo[:, _DH:]

    ul = ulse_ref[...]
    m = jnp.max(ul, axis=1, keepdims=True)
    w = jnp.exp(ul - m)
    wsum = jnp.sum(w, axis=1, keepdims=True)
    acc = jnp.zeros((_N, _DH), f32)
    for h in range(_NHASH):
        acc = acc + uo_ref[:, h * _DH:(h + 1) * _DH] * w[:, h:h + 1]
    o_ref[0] = acc / wsum


def _proj_body(x_ref, w_ref, b_ref, o_ref):
    o_ref[...] = _dot16(x_ref[...], w_ref[...], (((1,), (0,)), ((), ()))) + b_ref[...]


def kernel(x, Wqk, Wv, Wout, bout, rotations):
    B, C, H, W = x.shape
    n = 5 * C
    hw = (H // 2) * (W // 2)

    xc = pl.pallas_call(
        _frontend_body,
        grid=(B,),
        in_specs=[pl.BlockSpec((1, C, H, W), lambda b: (b, 0, 0, 0))],
        out_specs=pl.BlockSpec((1, n, H // 2, W // 2), lambda b: (b, 0, 0, 0)),
        out_shape=jax.ShapeDtypeStruct((B, n, H // 2, W // 2), jnp.float32),
    )(x)
    xf = xc.reshape(B, n, hw)

    rot2 = rotations.reshape(_DH, _NHASH * (_NB // 2))

    att = pl.pallas_call(
        _attn_body,
        grid=(B, _HEADS),
        in_specs=[
            pl.BlockSpec((1, _N, _DIM), lambda b, h: (b, 0, 0)),
            pl.BlockSpec((_DIM, _DH), lambda b, h: (0, h)),
            pl.BlockSpec((_DIM, _DH), lambda b, h: (0, h)),
            pl.BlockSpec((_DH, 40), lambda b, h: (0, 0)),
        ],
        out_specs=pl.BlockSpec((1, _N, _DH), lambda b, h: (b, 0, h)),
        out_shape=jax.ShapeDtypeStruct((B, _N, _DIM), jnp.float32),
        scratch_shapes=[
            pltpu.VMEM((_NSEQ, 2 * _DH), jnp.float32),
            pltpu.VMEM((_NSEQ, _DH), jnp.float32),
            pltpu.VMEM((_NSEQ, 1), jnp.float32),
            pltpu.VMEM((_NSEQ, _DH), jnp.float32),
            pltpu.VMEM((_NSEQ, 1), jnp.float32),
            pltpu.VMEM((_N, _NHASH), jnp.float32),
            pltpu.VMEM((_N, _NHASH * _DH), jnp.float32),
            pltpu.VMEM((_N, _NHASH), jnp.float32),
        ],
    )(xf, Wqk, Wv, rot2)

    y = pl.pallas_call(
        _proj_body,
        grid=(B * n // 256,),
        in_specs=[
            pl.BlockSpec((256, _DIM), lambda i: (i, 0)),
            pl.BlockSpec((_DIM, _DIM), lambda i: (0, 0)),
            pl.BlockSpec((1, _DIM), lambda i: (0, 0)),
        ],
        out_specs=pl.BlockSpec((256, _DIM), lambda i: (i, 0)),
        out_shape=jax.ShapeDtypeStruct((B * n, _DIM), jnp.float32),
    )(att.reshape(B * n, _DIM), Wout, bout.reshape(1, _DIM))

    return y.reshape(B, n, H // 2, W // 2)


# speedup vs baseline: 3.5866x; 2.1605x over previous
import jax
import jax.numpy as jnp
from jax import lax
from jax.experimental import pallas as pl
from jax.experimental.pallas import tpu as pltpu

_HEADS = 8
_BUCKET = 64
_NHASH = 8
_NB = 10
_N = 640
_DH = 128
_DIM = 1024
_NSEQ = _NHASH * _N
_NCH = _NSEQ // _BUCKET
_CPR = _N // _BUCKET

_HIGH = lax.Precision.HIGHEST
_DN = (((1,), (0,)), ((), ()))
_DNT = (((1,), (1,)), ((), ()))
_DNL = (((0,), (0,)), ((), ()))


def _dot16(a, b, dims):
    return lax.dot_general(a.astype(jnp.bfloat16), b.astype(jnp.bfloat16),
                           dims, preferred_element_type=jnp.float32)


def _dot16_2x(s, x, dims):
    sb = s.astype(jnp.bfloat16)
    xh = x.astype(jnp.bfloat16)
    xl = (x - xh.astype(jnp.float32)).astype(jnp.bfloat16)
    hi = lax.dot_general(sb, xh, dims, preferred_element_type=jnp.float32)
    lo = lax.dot_general(sb, xl, dims, preferred_element_type=jnp.float32)
    return hi + lo


def _frontend_body(x_ref, o_ref):
    f32 = jnp.float32
    x = x_ref[0]
    C, H, W = x.shape
    h2, w2 = H // 2, W // 2
    wi = lax.broadcasted_iota(jnp.int32, (W, w2), 0)
    ji = lax.broadcasted_iota(jnp.int32, (W, w2), 1)
    P0 = (wi == 2 * ji).astype(f32)
    P1 = (wi == 2 * ji + 1).astype(f32)

    ninf = jnp.full((C, 1, W), -jnp.inf, f32)
    up = jnp.concatenate([x[:, 1:, :], ninf], axis=1)
    dnm = jnp.concatenate([ninf, x[:, :-1, :]], axis=1)
    vm = jnp.maximum(x, jnp.maximum(up, dnm))
    vme = vm.reshape(C * h2, 2, W)[:, 0, :]
    ninfc = jnp.full((C * h2, 1), -jnp.inf, f32)
    lf = jnp.concatenate([vme[:, 1:], ninfc], axis=1)
    rt = jnp.concatenate([ninfc, vme[:, :-1]], axis=1)
    hm = jnp.maximum(vme, jnp.maximum(lf, rt))
    xs = lax.dot_general(hm, P0, _DN, precision=_HIGH)

    xr = x.reshape(C * h2, 2, W)
    xe = xr[:, 0, :]
    xo = xr[:, 1, :]
    x00 = lax.dot_general(xe, P0, _DN, precision=_HIGH)
    x01 = lax.dot_general(xe, P1, _DN, precision=_HIGH)
    x10 = lax.dot_general(xo, P0, _DN, precision=_HIGH)
    x11 = lax.dot_general(xo, P1, _DN, precision=_HIGH)
    ll = (x00 + x01 + x10 + x11) * 0.5
    lh = (x10 + x11 - x00 - x01) * 0.5
    hl = (x01 + x11 - x00 - x10) * 0.5
    hh = (x00 - x01 - x10 + x11) * 0.5
    o_ref[0] = jnp.concatenate(
        [t.reshape(C, h2, w2) for t in (xs, ll, lh, hl, hh)], axis=0)


def _attn_body(xf_ref, wqk_ref, wv_ref, rot_ref, o_ref,
               sx_ref, skn_ref, so_ref, lse_ref, uo_ref):
    f32 = jnp.float32
    xfb = xf_ref[0]
    qk = _dot16(xfb, wqk_ref[...], _DN)
    v = _dot16(xfb, wv_ref[...], _DN)
    X = jnp.concatenate([qk, v], axis=1)
    rot = _dot16(qk, rot_ref[...], _DN)

    ri = lax.broadcasted_iota(jnp.int32, (_N, _N), 0)
    ci = lax.broadcasted_iota(jnp.int32, (_N, _N), 1)
    Tlow = (ci <= ri).astype(f32)
    cif = ci.astype(f32)
    ti = lax.broadcasted_iota(jnp.int32, (_N, 1), 0).astype(f32)

    ki = lax.broadcasted_iota(jnp.int32, (_N, _NB), 1)
    j10 = lax.broadcasted_iota(jnp.int32, (_NB, _NB), 0)
    k10 = lax.broadcasted_iota(jnp.int32, (_NB, _NB), 1)
    Lstrict = (j10 < k10).astype(f32)

    pos_all = []
    idq_all = []
    idk_all = []
    for h in range(_NHASH):
        r5 = rot[:, h * 5:(h + 1) * 5]
        rh = jnp.concatenate([r5, -r5], axis=1)
        mx = jnp.max(rh, axis=1, keepdims=True)
        b = jnp.min(jnp.where(rh >= mx, ki, _NB), axis=1, keepdims=True)
        O = (ki == b).astype(f32)
        colcum = _dot16(Tlow, O, _DN)
        cnt = colcum[_N - 1:_N, :]
        offs = lax.dot_general(cnt, Lstrict, _DN, precision=_HIGH)
        rank = jnp.sum(O * colcum, axis=1, keepdims=True) - 1.0
        base = lax.dot_general(O, offs, _DNT, precision=_HIGH)
        pos = base + rank
        pos_all.append(pos)
        S = (pos == cif).astype(f32)
        sx = _dot16_2x(S, X, _DNL)
        idq_all.append(
            lax.dot_general(S[:, :_BUCKET], ti, _DNL, precision=_HIGH))
        idk_all.append(
            lax.dot_general(S[:, _N - _BUCKET:], ti, _DNL, precision=_HIGH))
        sk = sx[:, :_DH]
        nrm = jnp.sqrt(jnp.sum(sk * sk, axis=1, keepdims=True))
        kn = sk / jnp.maximum(nrm, 1e-12)
        sx_ref[h * _N:(h + 1) * _N, :] = sx
        skn_ref[h * _N:(h + 1) * _N, :] = kn

    scale = _DH ** -0.5
    qi = lax.broadcasted_iota(jnp.int32, (_BUCKET, _BUCKET), 0)
    kj = lax.broadcasted_iota(jnp.int32, (_BUCKET, _BUCKET), 1)
    eye = qi == kj
    ones_q = jnp.ones((_BUCKET, 1), f32)
    ones_k = jnp.ones((_BUCKET, 1), f32)

    for c in range(_NCH):
        pc = (c - 1) % _NCH
        cur = sx_ref[c * _BUCKET:(c + 1) * _BUCKET, :]
        q = cur[:, :_DH]
        kc = skn_ref[c * _BUCKET:(c + 1) * _BUCKET, :]
        kp = skn_ref[pc * _BUCKET:(pc + 1) * _BUCKET, :]
        kk = jnp.concatenate([kc, kp], axis=0)
        prev = sx_ref[pc * _BUCKET:(pc + 1) * _BUCKET, :]
        vv = jnp.concatenate([cur[:, _DH:], prev[:, _DH:]], axis=0)
        dots = _dot16(q, kk, _DNT) * scale
        d1 = jnp.where(eye, -5e4, dots[:, :_BUCKET])
        d2 = dots[:, _BUCKET:]
        if c % _CPR == 0:
            h = c // _CPR
            idq = idq_all[h]
            idk = idk_all[(h - 1) % _NHASH]
            diff = lax.dot_general(
                jnp.concatenate([idq, ones_q], axis=1),
                jnp.concatenate([ones_k, -idk], axis=1),
                _DNT, precision=_HIGH)
            d2 = jnp.where(diff == 0.0, -5e4, d2)
        dots = jnp.concatenate([d1, d2], axis=1)
        m = jnp.max(dots, axis=1, keepdims=True)
        ex = jnp.exp(dots - m)
        s = jnp.sum(ex, axis=1, keepdims=True)
        so = _dot16(ex / s, vv, _DN)
        so_ref[c * _BUCKET:(c + 1) * _BUCKET, :] = so
        lse_ref[c * _BUCKET:(c + 1) * _BUCKET, :] = m + jnp.log(s)

    ulse_all = []
    for h in range(_NHASH):
        U = (pos_all[h] == cif).astype(f32)
        uo = _dot16_2x(U, so_ref[h * _N:(h + 1) * _N, :], _DN)
        ulse_all.append(_dot16_2x(U, lse_ref[h * _N:(h + 1) * _N, :], _DN))
        uo_ref[:, h * _DH:(h + 1) * _DH] = uo

    ul = jnp.concatenate(ulse_all, axis=1)
    m = jnp.max(ul, axis=1, keepdims=True)
    w = jnp.exp(ul - m)
    wsum = jnp.sum(w, axis=1, keepdims=True)
    acc = jnp.zeros((_N, _DH), f32)
    for h in range(_NHASH):
        acc = acc + uo_ref[:, h * _DH:(h + 1) * _DH] * w[:, h:h + 1]
    o_ref[0] = acc / wsum


def _proj_body(x_ref, w_ref, b_ref, o_ref):
    o_ref[...] = _dot16(x_ref[...], w_ref[...], _DN) + b_ref[...]


def kernel(x, Wqk, Wv, Wout, bout, rotations):
    B, C, H, W = x.shape
    n = 5 * C
    hw = (H // 2) * (W // 2)

    xc = pl.pallas_call(
        _frontend_body,
        grid=(B,),
        in_specs=[pl.BlockSpec((1, C, H, W), lambda b: (b, 0, 0, 0))],
        out_specs=pl.BlockSpec((1, n, H // 2, W // 2), lambda b: (b, 0, 0, 0)),
        out_shape=jax.ShapeDtypeStruct((B, n, H // 2, W // 2), jnp.float32),
    )(x)
    xf = xc.reshape(B, n, hw)

    rot2 = rotations.reshape(_DH, _NHASH * (_NB // 2))

    att = pl.pallas_call(
        _attn_body,
        grid=(B, _HEADS),
        in_specs=[
            pl.BlockSpec((1, _N, _DIM), lambda b, h: (b, 0, 0)),
            pl.BlockSpec((_DIM, _DH), lambda b, h: (0, h)),
            pl.BlockSpec((_DIM, _DH), lambda b, h: (0, h)),
            pl.BlockSpec((_DH, 40), lambda b, h: (0, 0)),
        ],
        out_specs=pl.BlockSpec((1, _N, _DH), lambda b, h: (b, 0, h)),
        out_shape=jax.ShapeDtypeStruct((B, _N, _DIM), jnp.float32),
        scratch_shapes=[
            pltpu.VMEM((_NSEQ, 2 * _DH), jnp.float32),
            pltpu.VMEM((_NSEQ, _DH), jnp.float32),
            pltpu.VMEM((_NSEQ, _DH), jnp.float32),
            pltpu.VMEM((_NSEQ, 1), jnp.float32),
            pltpu.VMEM((_N, _NHASH * _DH), jnp.float32),
        ],
    )(xf, Wqk, Wv, rot2)

    y = pl.pallas_call(
        _proj_body,
        grid=(B * n // 256,),
        in_specs=[
            pl.BlockSpec((256, _DIM), lambda i: (i, 0)),
            pl.BlockSpec((_DIM, _DIM), lambda i: (0, 0)),
            pl.BlockSpec((1, _DIM), lambda i: (0, 0)),
        ],
        out_specs=pl.BlockSpec((256, _DIM), lambda i: (i, 0)),
        out_shape=jax.ShapeDtypeStruct((B * n, _DIM), jnp.float32),
    )(att.reshape(B * n, _DIM), Wout, bout.reshape(1, _DIM))

    return y.reshape(B, n, H // 2, W // 2)


# speedup vs baseline: 3.8250x; 1.0665x over previous
import jax
import jax.numpy as jnp
from jax import lax
from jax.experimental import pallas as pl
from jax.experimental.pallas import tpu as pltpu

_HEADS = 8
_BUCKET = 64
_NHASH = 8
_NB = 10
_N = 640
_DH = 128
_DIM = 1024
_NSEQ = _NHASH * _N
_NCH = _NSEQ // _BUCKET
_CPR = _N // _BUCKET

_HIGH = lax.Precision.HIGHEST
_DN = (((1,), (0,)), ((), ()))
_DNT = (((1,), (1,)), ((), ()))
_DNL = (((0,), (0,)), ((), ()))
_BF = jnp.bfloat16


def _dot16(a, b, dims):
    return lax.dot_general(a.astype(_BF), b.astype(_BF),
                           dims, preferred_element_type=jnp.float32)


def _split(x):
    xh = x.astype(_BF)
    xl = (x - xh.astype(jnp.float32)).astype(_BF)
    return xh, xl


def _dot2x(sb, xh, xl, dims):
    hi = lax.dot_general(sb, xh, dims, preferred_element_type=jnp.float32)
    lo = lax.dot_general(sb, xl, dims, preferred_element_type=jnp.float32)
    return hi + lo


def _onehot16(pos, cif):
    return (pos == cif).astype(jnp.float32).astype(_BF)


def _frontend_body(x_ref, o_ref):
    f32 = jnp.float32
    x = x_ref[0]
    C, H, W = x.shape
    h2, w2 = H // 2, W // 2
    wi = lax.broadcasted_iota(jnp.int32, (W, w2), 0)
    ji = lax.broadcasted_iota(jnp.int32, (W, w2), 1)
    P0 = (wi == 2 * ji).astype(f32)
    P1 = (wi == 2 * ji + 1).astype(f32)

    ninf = jnp.full((C, 1, W), -jnp.inf, f32)
    up = jnp.concatenate([x[:, 1:, :], ninf], axis=1)
    dnm = jnp.concatenate([ninf, x[:, :-1, :]], axis=1)
    vm = jnp.maximum(x, jnp.maximum(up, dnm))
    vme = vm.reshape(C * h2, 2, W)[:, 0, :]
    ninfc = jnp.full((C * h2, 1), -jnp.inf, f32)
    lf = jnp.concatenate([vme[:, 1:], ninfc], axis=1)
    rt = jnp.concatenate([ninfc, vme[:, :-1]], axis=1)
    hm = jnp.maximum(vme, jnp.maximum(lf, rt))
    xs = lax.dot_general(hm, P0, _DN, precision=_HIGH)

    xr = x.reshape(C * h2, 2, W)
    xe = xr[:, 0, :]
    xo = xr[:, 1, :]
    x00 = lax.dot_general(xe, P0, _DN, precision=_HIGH)
    x01 = lax.dot_general(xe, P1, _DN, precision=_HIGH)
    x10 = lax.dot_general(xo, P0, _DN, precision=_HIGH)
    x11 = lax.dot_general(xo, P1, _DN, precision=_HIGH)
    ll = (x00 + x01 + x10 + x11) * 0.5
    lh = (x10 + x11 - x00 - x01) * 0.5
    hl = (x01 + x11 - x00 - x10) * 0.5
    hh = (x00 - x01 - x10 + x11) * 0.5
    o_ref[0] = jnp.concatenate(
        [t.reshape(C, h2, w2) for t in (xs, ll, lh, hl, hh)], axis=0)


def _attn_body(xf_ref, wqk_ref, wv_ref, rot_ref, o_ref,
               sq_ref, sv_ref, skn_ref, so_ref, lse_ref, uo_ref):
    f32 = jnp.float32
    xfb = xf_ref[0]
    qk = _dot16(xfb, wqk_ref[...], _DN)
    v = _dot16(xfb, wv_ref[...], _DN)
    X = jnp.concatenate([qk, v], axis=1)
    rot = _dot16(qk, rot_ref[...], _DN)
    Xh, Xl = _split(X)

    ri = lax.broadcasted_iota(jnp.int32, (_N, _N), 0)
    ci = lax.broadcasted_iota(jnp.int32, (_N, _N), 1)
    Tlow = (ci <= ri).astype(jnp.float32).astype(_BF)
    cif = ci.astype(f32)
    ti = lax.broadcasted_iota(jnp.int32, (_N, 1), 0).astype(f32)
    tih, til = _split(ti)

    ki = lax.broadcasted_iota(jnp.int32, (_N, _NB), 1)

    pos_all = []
    idq_all = []
    idk_all = []
    for h in range(_NHASH):
        r5 = rot[:, h * 5:(h + 1) * 5]
        rh = jnp.concatenate([r5, -r5], axis=1)
        mx = jnp.max(rh, axis=1, keepdims=True)
        b = jnp.min(jnp.where(rh >= mx, ki, _NB), axis=1, keepdims=True)
        O = (ki == b).astype(f32)
        colcum = lax.dot_general(Tlow, O.astype(_BF), _DN,
                                 preferred_element_type=f32)
        cnt = colcum[_N - 1:_N, :]
        z = jnp.concatenate([jnp.zeros((1, 1), f32), cnt[:, :-1]], axis=1)
        for sh in (1, 2, 4, 8):
            z = z + jnp.concatenate(
                [jnp.zeros((1, sh), f32), z[:, :-sh]], axis=1)
        offs = z
        rank = jnp.sum(O * colcum, axis=1, keepdims=True) - 1.0
        base = jnp.sum(O * offs, axis=1, keepdims=True)
        pos = base + rank
        pos_all.append(pos)
        S = _onehot16(pos, cif)
        sx = _dot2x(S, Xh, Xl, _DNL)
        idq_all.append(_dot2x(S[:, :_BUCKET], tih, til, _DNL))
        idk_all.append(_dot2x(S[:, _N - _BUCKET:], tih, til, _DNL))
        sk = sx[:, :_DH]
        nrm = jnp.sqrt(jnp.sum(sk * sk, axis=1, keepdims=True))
        kn = sk / jnp.maximum(nrm, 1e-12)
        sq_ref[h * _N:(h + 1) * _N, :] = sk.astype(_BF)
        sv_ref[h * _N:(h + 1) * _N, :] = sx[:, _DH:].astype(_BF)
        skn_ref[h * _N:(h + 1) * _N, :] = kn.astype(_BF)

    scale = _DH ** -0.5
    qi = lax.broadcasted_iota(jnp.int32, (_BUCKET, _BUCKET), 0)
    kj = lax.broadcasted_iota(jnp.int32, (_BUCKET, _BUCKET), 1)
    eye = qi == kj
    ones_q = jnp.ones((_BUCKET, 1), f32)
    ones_k = jnp.ones((_BUCKET, 1), f32)

    for c in range(_NCH):
        pc = (c - 1) % _NCH
        q = sq_ref[c * _BUCKET:(c + 1) * _BUCKET, :]
        kc = skn_ref[c * _BUCKET:(c + 1) * _BUCKET, :]
        kp = skn_ref[pc * _BUCKET:(pc + 1) * _BUCKET, :]
        kk = jnp.concatenate([kc, kp], axis=0)
        vv = jnp.concatenate([sv_ref[c * _BUCKET:(c + 1) * _BUCKET, :],
                              sv_ref[pc * _BUCKET:(pc + 1) * _BUCKET, :]],
                             axis=0)
        dots = lax.dot_general(q, kk, _DNT,
                               preferred_element_type=f32) * scale
        d1 = jnp.where(eye, -5e4, dots[:, :_BUCKET])
        d2 = dots[:, _BUCKET:]
        if c % _CPR == 0:
            h = c // _CPR
            idq = idq_all[h]
            idk = idk_all[(h - 1) % _NHASH]
            diff = lax.dot_general(
                jnp.concatenate([idq, ones_q], axis=1),
                jnp.concatenate([ones_k, -idk], axis=1),
                _DNT, precision=_HIGH)
            d2 = jnp.where(diff == 0.0, -5e4, d2)
        dots = jnp.concatenate([d1, d2], axis=1)
        m = jnp.max(dots, axis=1, keepdims=True)
        ex = jnp.exp(dots - m)
        s = jnp.sum(ex, axis=1, keepdims=True)
        so = lax.dot_general((ex / s).astype(_BF), vv, _DN,
                             preferred_element_type=f32)
        so_ref[c * _BUCKET:(c + 1) * _BUCKET, :] = so
        lse_ref[c * _BUCKET:(c + 1) * _BUCKET, :] = m + jnp.log(s)

    ulse_all = []
    for h in range(_NHASH):
        U = _onehot16(pos_all[h], cif)
        seg = jnp.concatenate([so_ref[h * _N:(h + 1) * _N, :],
                               lse_ref[h * _N:(h + 1) * _N, :]], axis=1)
        segh, segl = _split(seg)
        uo = _dot2x(U, segh, segl, _DN)
        ulse_all.append(uo[:, _DH:])
        uo_ref[:, h * _DH:(h + 1) * _DH] = uo[:, :_DH]

    ul = jnp.concatenate(ulse_all, axis=1)
    m = jnp.max(ul, axis=1, keepdims=True)
    w = jnp.exp(ul - m)
    wsum = jnp.sum(w, axis=1, keepdims=True)
    acc = jnp.zeros((_N, _DH), f32)
    for h in range(_NHASH):
        acc = acc + uo_ref[:, h * _DH:(h + 1) * _DH] * w[:, h:h + 1]
    o_ref[0] = acc / wsum


def _proj_body(x_ref, w_ref, b_ref, o_ref):
    o_ref[...] = _dot16(x_ref[...], w_ref[...], _DN) + b_ref[...]


def kernel(x, Wqk, Wv, Wout, bout, rotations):
    B, C, H, W = x.shape
    n = 5 * C
    hw = (H // 2) * (W // 2)

    xc = pl.pallas_call(
        _frontend_body,
        grid=(B,),
        in_specs=[pl.BlockSpec((1, C, H, W), lambda b: (b, 0, 0, 0))],
        out_specs=pl.BlockSpec((1, n, H // 2, W // 2), lambda b: (b, 0, 0, 0)),
        out_shape=jax.ShapeDtypeStruct((B, n, H // 2, W // 2), jnp.float32),
    )(x)
    xf = xc.reshape(B, n, hw)

    rot2 = rotations.reshape(_DH, _NHASH * (_NB // 2))

    att = pl.pallas_call(
        _attn_body,
        grid=(B, _HEADS),
        in_specs=[
            pl.BlockSpec((1, _N, _DIM), lambda b, h: (b, 0, 0)),
            pl.BlockSpec((_DIM, _DH), lambda b, h: (0, h)),
            pl.BlockSpec((_DIM, _DH), lambda b, h: (0, h)),
            pl.BlockSpec((_DH, 40), lambda b, h: (0, 0)),
        ],
        out_specs=pl.BlockSpec((1, _N, _DH), lambda b, h: (b, 0, h)),
        out_shape=jax.ShapeDtypeStruct((B, _N, _DIM), jnp.float32),
        scratch_shapes=[
            pltpu.VMEM((_NSEQ, _DH), _BF),
            pltpu.VMEM((_NSEQ, _DH), _BF),
            pltpu.VMEM((_NSEQ, _DH), _BF),
            pltpu.VMEM((_NSEQ, _DH), jnp.float32),
            pltpu.VMEM((_NSEQ, 1), jnp.float32),
            pltpu.VMEM((_N, _NHASH * _DH), jnp.float32),
        ],
    )(xf, Wqk, Wv, rot2)

    y = pl.pallas_call(
        _proj_body,
        grid=(B * n // 256,),
        in_specs=[
            pl.BlockSpec((256, _DIM), lambda i: (i, 0)),
            pl.BlockSpec((_DIM, _DIM), lambda i: (0, 0)),
            pl.BlockSpec((1, _DIM), lambda i: (0, 0)),
        ],
        out_specs=pl.BlockSpec((256, _DIM), lambda i: (i, 0)),
        out_shape=jax.ShapeDtypeStruct((B * n, _DIM), jnp.float32),
    )(att.reshape(B * n, _DIM), Wout, bout.reshape(1, _DIM))

    return y.reshape(B, n, H // 2, W // 2)
